# baseline (speedup 1.0000x reference)
import jax
import jax.numpy as jnp
from jax.experimental import pallas as pl
from jax.experimental.pallas import tpu as pltpu


def _dma_copy(src_ref, out_ref, sem):
    copy = pltpu.make_async_copy(src_ref.at[0:1], out_ref, sem)
    copy.start()
    copy.wait()


def kernel(last_hidden_state, input_ids):
    del input_ids
    B, S, H = last_hidden_state.shape
    out = pl.pallas_call(
        _dma_copy,
        in_specs=[pl.BlockSpec(memory_space=pl.ANY)],
        out_specs=pl.BlockSpec(memory_space=pl.ANY),
        out_shape=jax.ShapeDtypeStruct((1, S, H), last_hidden_state.dtype),
        scratch_shapes=[pltpu.SemaphoreType.DMA],
    )(last_hidden_state)
    return out
